# Initial kernel scaffold; baseline (speedup 1.0000x reference)
import jax
import jax.numpy as jnp
from jax.experimental import pallas as pl


def kernel(x, edge_index, params):
    raise NotImplementedError("write your pallas kernel here")


# speedup vs baseline: 5.2233x; 5.2233x over previous
import dataclasses
import functools

import jax
import jax.numpy as jnp
from jax import lax
from jax.experimental import pallas as pl
from jax.experimental.pallas import tpu as pltpu
from jax.experimental.pallas import tpu_sc as plsc

N = 10000
E = 320000
H = 128
NC = 2
NS = 16
NW = NC * NS
EPT = E // NW
CHUNK = 128
NFULL = EPT // CHUNK
REM = EPT - NFULL * CHUNK
SLAB = 640
LAST_SLAB = N - (NS - 1) * SLAB

@functools.cache
def _sc_kernels():
    mesh = plsc.VectorSubcoreMesh(core_axis_name="c", subcore_axis_name="s")
    cp = pltpu.CompilerParams()
    if "needs_layout_passes" in pltpu.CompilerParams.__dataclass_fields__:
        cp = dataclasses.replace(cp, needs_layout_passes=False)

    @functools.partial(
        pl.kernel,
        out_type=jax.ShapeDtypeStruct((NW, N), jnp.float32),
        mesh=mesh,
        compiler_params=cp,
        scratch_types=[
            pltpu.VMEM((EPT,), jnp.int32),
            pltpu.VMEM((N,), jnp.float32),
        ],
    )
    def sc_degree(row_hbm, out_hbm, idx_v, hist_v):
        c = lax.axis_index("c")
        s = lax.axis_index("s")
        g = c * NS + s
        pltpu.sync_copy(row_hbm.at[pl.ds(g * EPT, EPT)], idx_v)
        zeros16 = jnp.zeros((16,), jnp.float32)

        @pl.loop(0, N // 16)
        def _(i):
            hist_v[pl.ds(i * 16, 16)] = zeros16

        ones16 = jnp.ones((16,), jnp.float32)

        @pl.loop(0, EPT // 16)
        def _(j):
            idx = idx_v[pl.ds(j * 16, 16)]
            plsc.addupdate_scatter(hist_v, [idx], ones16)

        pltpu.sync_copy(hist_v, out_hbm.at[g])

    @functools.partial(
        pl.kernel,
        out_type=jax.ShapeDtypeStruct((NC, N, H), jnp.float32),
        mesh=mesh,
        scratch_types=[
            pltpu.VMEM((CHUNK,), jnp.int32),
            pltpu.VMEM((CHUNK,), jnp.int32),
            pltpu.VMEM((CHUNK, H), jnp.float32),
            pltpu.VMEM((REM,), jnp.int32),
            pltpu.VMEM((REM,), jnp.int32),
            pltpu.VMEM((REM, H), jnp.float32),
            pltpu.VMEM_SHARED((N, H), jnp.float32),
            pltpu.SemaphoreType.DMA,
        ],
    )
    def sc_aggregate(h_hbm, col_hbm, row_hbm, z_hbm, out_hbm,
                     ci_v, ri_v, rows_v, ci16_v, ri16_v, rows16_v, agg_sh, sem):
        c = lax.axis_index("c")
        s = lax.axis_index("s")
        g = c * NS + s

        @pl.when(s < NS - 1)
        def _():
            pltpu.sync_copy(z_hbm.at[pl.ds(s * SLAB, SLAB)],
                            agg_sh.at[pl.ds(s * SLAB, SLAB)])

        @pl.when(s == NS - 1)
        def _():
            pltpu.sync_copy(z_hbm.at[pl.ds((NS - 1) * SLAB, LAST_SLAB)],
                            agg_sh.at[pl.ds((NS - 1) * SLAB, LAST_SLAB)])

        plsc.subcore_barrier()

        base = g * EPT

        @pl.loop(0, NFULL)
        def _(j):
            off = base + j * CHUNK
            pltpu.sync_copy(col_hbm.at[pl.ds(off, CHUNK)], ci_v)
            pltpu.async_copy(h_hbm.at[ci_v], rows_v, sem).wait()
            pltpu.sync_copy(row_hbm.at[pl.ds(off, CHUNK)], ri_v)
            pltpu.sync_copy(rows_v, agg_sh.at[ri_v], add=True)

        off = base + NFULL * CHUNK
        pltpu.sync_copy(col_hbm.at[pl.ds(off, REM)], ci16_v)
        pltpu.async_copy(h_hbm.at[ci16_v], rows16_v, sem).wait()
        pltpu.sync_copy(row_hbm.at[pl.ds(off, REM)], ri16_v)
        pltpu.sync_copy(rows16_v, agg_sh.at[ri16_v], add=True)

        plsc.subcore_barrier()

        @pl.when(s < NS - 1)
        def _():
            pltpu.sync_copy(agg_sh.at[pl.ds(s * SLAB, SLAB)],
                            out_hbm.at[c].at[pl.ds(s * SLAB, SLAB)])

        @pl.when(s == NS - 1)
        def _():
            pltpu.sync_copy(agg_sh.at[pl.ds((NS - 1) * SLAB, LAST_SLAB)],
                            out_hbm.at[c].at[pl.ds((NS - 1) * SLAB, LAST_SLAB)])

    return sc_degree, sc_aggregate


def _dot(a, b):
    return lax.dot_general(a, b, (((1,), (0,)), ((), ())),
                           precision=lax.Precision.HIGHEST,
                           preferred_element_type=jnp.float32)


def _tc_input_body(x_ref, w_ref, b_ref, pt_ref, h_ref, rdeg_ref):
    h_ref[...] = jnp.maximum(_dot(x_ref[...], w_ref[...]) + b_ref[...], 0.0)
    d = jnp.sum(pt_ref[...], axis=1, keepdims=True)
    rdeg_ref[...] = 1.0 / jnp.maximum(d, 1.0)


def _tc_input(x, w, b, pt):
    return pl.pallas_call(
        _tc_input_body,
        out_shape=(
            jax.ShapeDtypeStruct((N, H), jnp.float32),
            jax.ShapeDtypeStruct((N, 1), jnp.float32),
        ),
    )(x, w, b, pt)


def _tc_layer_body(h_ref, a0_ref, a1_ref, rdeg_ref, w_ref, b_ref,
                   gamma_ref, beta_ref, o_ref):
    t = h_ref[...] + (a0_ref[...] + a1_ref[...]) * rdeg_ref[...]
    y = _dot(t, w_ref[...]) + b_ref[...]
    mean = jnp.mean(y, axis=0, keepdims=True)
    cen = y - mean
    var = jnp.mean(cen * cen, axis=0, keepdims=True)
    o_ref[...] = jnp.maximum(
        cen * lax.rsqrt(var + 1e-5) * gamma_ref[...] + beta_ref[...], 0.0)


def _tc_layer(h, a0, a1, rdeg, w, b, gamma, beta):
    return pl.pallas_call(
        _tc_layer_body,
        out_shape=jax.ShapeDtypeStruct((N, H), jnp.float32),
    )(h, a0, a1, rdeg, w, b, gamma, beta)


def _tc_head_body(h_ref, wp_ref, bp_ref, wh_ref, bh_ref, o_ref):
    g = jnp.mean(h_ref[...], axis=0, keepdims=True)
    z = jnp.maximum(_dot(g, wp_ref[...]) + bp_ref[...], 0.0)
    o_ref[...] = jax.nn.sigmoid(_dot(z, wh_ref[...]) + bh_ref[...])


def _tc_head(h, wp, bp, wh, bh):
    return pl.pallas_call(
        _tc_head_body,
        out_shape=jax.ShapeDtypeStruct((1, 9), jnp.float32),
    )(h, wp, bp, wh, bh)


def kernel(x, edge_index, params):
    row = edge_index[0]
    col = edge_index[1]
    sc_degree, sc_aggregate = _sc_kernels()

    deg_p = sc_degree(row)
    pt = deg_p.T
    h, rdeg = _tc_input(x, params["W_in"], params["b_in"].reshape(1, H), pt)

    zinit = jnp.zeros((N, H), jnp.float32)
    for lp in params["layers"]:
        agg2 = sc_aggregate(h, col, row, zinit)
        h = _tc_layer(h, agg2[0], agg2[1], rdeg,
                      lp["W"], lp["b"].reshape(1, H),
                      lp["gamma"].reshape(1, H), lp["beta"].reshape(1, H))

    wh = jnp.concatenate([hp["W"] for hp in params["heads"]], axis=1)
    bh = jnp.concatenate([hp["b"] for hp in params["heads"]]).reshape(1, 9)
    return _tc_head(h, params["W_pool"], params["b_pool"].reshape(1, H), wh, bh)
